# baseline (speedup 1.0000x reference)
import functools

import jax
import jax.numpy as jnp
from jax import lax
from jax.experimental import pallas as pl
from jax.experimental.pallas import tpu as pltpu
from jax.experimental.pallas import tpu_sc as plsc

_NC = 2
_NS = 16
_L = 16
_NW = _NC * _NS
_K = 128
_BLK = 10240


def _sc_degree(dst_idx, n_pad, n_chunks):
  zrows = n_pad // _NS
  mesh = plsc.VectorSubcoreMesh(core_axis_name="c", subcore_axis_name="s")

  @functools.partial(
      pl.kernel,
      out_type=jax.ShapeDtypeStruct((_NC, n_pad), jnp.float32),
      mesh=mesh,
      scratch_types=[
          pltpu.VMEM((n_chunks, _K), jnp.int32),
          pltpu.VMEM((_K,), jnp.float32),
          pltpu.VMEM_SHARED((n_pad,), jnp.float32),
      ],
  )
  def k(dst_hbm, out_hbm, di_v, ones_v, deg_sh):
    cid = lax.axis_index("c")
    sid = lax.axis_index("s")
    wid = cid * _NS + sid
    pltpu.sync_copy(dst_hbm.at[wid], di_v)

    def _fill(val, i, carry):
      ones_v[pl.ds(i * _L, _L)] = jnp.full((_L,), val, jnp.float32)
      return carry

    lax.fori_loop(0, _K // _L, functools.partial(_fill, 0.0), 0)
    for z in range(zrows // _K):
      pltpu.sync_copy(ones_v, deg_sh.at[pl.ds(sid * zrows + z * _K, _K)])
    lax.fori_loop(0, _K // _L, functools.partial(_fill, 1.0), 0)
    plsc.subcore_barrier()

    def chunk(j, carry):
      pltpu.sync_copy(ones_v, deg_sh.at[di_v.at[j]], add=True)
      return carry

    lax.fori_loop(0, n_chunks, chunk, 0)
    plsc.subcore_barrier()
    pltpu.sync_copy(deg_sh.at[pl.ds(sid * zrows, zrows)],
                    out_hbm.at[cid, pl.ds(sid * zrows, zrows)])

  return k(dst_idx)


def _sc_scatter(y, src_idx, dst_idx, n_pad, d, n_chunks):
  zrows = n_pad // _NS
  mesh = plsc.VectorSubcoreMesh(core_axis_name="c", subcore_axis_name="s")

  @functools.partial(
      pl.kernel,
      out_type=jax.ShapeDtypeStruct((_NC, n_pad, d), jnp.float32),
      mesh=mesh,
      scratch_types=[
          pltpu.VMEM((n_chunks, _K), jnp.int32),
          pltpu.VMEM((n_chunks, _K), jnp.int32),
          pltpu.VMEM((_K, d), jnp.float32),
          pltpu.SemaphoreType.DMA,
          pltpu.VMEM_SHARED((n_pad, d), jnp.float32),
      ],
  )
  def k(y_hbm, src_hbm, dst_hbm, out_hbm, si_v, di_v, buf, gsem, s_sh):
    cid = lax.axis_index("c")
    sid = lax.axis_index("s")
    wid = cid * _NS + sid
    pltpu.sync_copy(src_hbm.at[wid], si_v)
    pltpu.sync_copy(dst_hbm.at[wid], di_v)

    def _zero(i, carry):
      for j in range(d // _L):
        buf[i, pl.ds(j * _L, _L)] = jnp.zeros((_L,), jnp.float32)
      return carry

    lax.fori_loop(0, _K, _zero, 0)
    for z in range(zrows // _K):
      pltpu.sync_copy(buf, s_sh.at[pl.ds(sid * zrows + z * _K, _K)])
    plsc.subcore_barrier()

    def chunk(j, carry):
      pltpu.async_copy(y_hbm.at[si_v.at[j]], buf, gsem).wait()
      pltpu.sync_copy(buf, s_sh.at[di_v.at[j]], add=True)
      return carry

    lax.fori_loop(0, n_chunks, chunk, 0)
    plsc.subcore_barrier()
    pltpu.sync_copy(s_sh.at[pl.ds(sid * zrows, zrows)],
                    out_hbm.at[cid, pl.ds(sid * zrows, zrows)])

  return k(y, src_idx, dst_idx)


def _tc_y1(x, w1, deg, n_pad, d_in, d_hid):
  def body(x_ref, w_ref, dg_ref, y_ref, di_ref):
    dg = dg_ref[...]
    dinv = lax.rsqrt(dg[0] + dg[1] + 1.0)
    di_ref[...] = dinv
    xw = jnp.dot(x_ref[...], w_ref[...], preferred_element_type=jnp.float32)
    y_ref[...] = xw * dinv

  return pl.pallas_call(
      body,
      grid=(n_pad // _BLK,),
      in_specs=[
          pl.BlockSpec((_BLK, d_in), lambda i: (i, 0)),
          pl.BlockSpec((d_in, d_hid), lambda i: (0, 0)),
          pl.BlockSpec((_NC, _BLK, 1), lambda i: (0, i, 0)),
      ],
      out_specs=[
          pl.BlockSpec((_BLK, d_hid), lambda i: (i, 0)),
          pl.BlockSpec((_BLK, 1), lambda i: (i, 0)),
      ],
      out_shape=[
          jax.ShapeDtypeStruct((n_pad, d_hid), jnp.float32),
          jax.ShapeDtypeStruct((n_pad, 1), jnp.float32),
      ],
  )(x, w1, deg)


def _tc_combine_stats(s, y1, dinv, b1, n_real, n_pad, d_hid):
  def body(s_ref, y_ref, di_ref, b_ref, h_ref, sum_ref, sq_ref):
    i = pl.program_id(0)
    h = di_ref[...] * (jnp.sum(s_ref[...], axis=0) + y_ref[...]) + b_ref[...]
    h_ref[...] = h
    rows = i * _BLK + lax.broadcasted_iota(jnp.int32, (_BLK, 1), 0)
    hm = jnp.where(rows < n_real, h, 0.0)

    @pl.when(i == 0)
    def _():
      sum_ref[...] = jnp.zeros_like(sum_ref)
      sq_ref[...] = jnp.zeros_like(sq_ref)

    sum_ref[...] += jnp.sum(hm, axis=0, keepdims=True)
    sq_ref[...] += jnp.sum(hm * hm, axis=0, keepdims=True)

  return pl.pallas_call(
      body,
      grid=(n_pad // _BLK,),
      in_specs=[
          pl.BlockSpec((_NC, _BLK, d_hid), lambda i: (0, i, 0)),
          pl.BlockSpec((_BLK, d_hid), lambda i: (i, 0)),
          pl.BlockSpec((_BLK, 1), lambda i: (i, 0)),
          pl.BlockSpec((1, d_hid), lambda i: (0, 0)),
      ],
      out_specs=[
          pl.BlockSpec((_BLK, d_hid), lambda i: (i, 0)),
          pl.BlockSpec((1, d_hid), lambda i: (0, 0)),
          pl.BlockSpec((1, d_hid), lambda i: (0, 0)),
      ],
      out_shape=[
          jax.ShapeDtypeStruct((n_pad, d_hid), jnp.float32),
          jax.ShapeDtypeStruct((1, d_hid), jnp.float32),
          jax.ShapeDtypeStruct((1, d_hid), jnp.float32),
      ],
  )(s, y1, dinv, b1)


def _tc_bn_relu_y2(h_pre, ssum, ssq, gamma, beta, w2, dinv, n_real, n_pad,
                   d_hid, d_out):
  def body(h_ref, sum_ref, sq_ref, g_ref, bt_ref, w_ref, di_ref, y_ref):
    mean = sum_ref[...] * (1.0 / n_real)
    var = sq_ref[...] * (1.0 / n_real) - mean * mean
    scale = lax.rsqrt(var + 1e-5) * g_ref[...]
    h = (h_ref[...] - mean) * scale + bt_ref[...]
    h = jnp.maximum(h, 0.0)
    hw = jnp.dot(h, w_ref[...], preferred_element_type=jnp.float32)
    y_ref[...] = hw * di_ref[...]

  return pl.pallas_call(
      body,
      grid=(n_pad // _BLK,),
      in_specs=[
          pl.BlockSpec((_BLK, d_hid), lambda i: (i, 0)),
          pl.BlockSpec((1, d_hid), lambda i: (0, 0)),
          pl.BlockSpec((1, d_hid), lambda i: (0, 0)),
          pl.BlockSpec((1, d_hid), lambda i: (0, 0)),
          pl.BlockSpec((1, d_hid), lambda i: (0, 0)),
          pl.BlockSpec((d_hid, d_out), lambda i: (0, 0)),
          pl.BlockSpec((_BLK, 1), lambda i: (i, 0)),
      ],
      out_specs=pl.BlockSpec((_BLK, d_out), lambda i: (i, 0)),
      out_shape=jax.ShapeDtypeStruct((n_pad, d_out), jnp.float32),
  )(h_pre, ssum, ssq, gamma, beta, w2, dinv)


def _tc_finish(s2, y2, dinv, b2, n_pad, d2, d_out):
  def body(s_ref, y_ref, di_ref, b_ref, o_ref):
    full = di_ref[...] * (jnp.sum(s_ref[...], axis=0) + y_ref[...])
    o_ref[...] = full[:, :d_out] + b_ref[...]

  return pl.pallas_call(
      body,
      grid=(n_pad // _BLK,),
      in_specs=[
          pl.BlockSpec((_NC, _BLK, d2), lambda i: (0, i, 0)),
          pl.BlockSpec((_BLK, d2), lambda i: (i, 0)),
          pl.BlockSpec((_BLK, 1), lambda i: (i, 0)),
          pl.BlockSpec((1, d_out), lambda i: (0, 0)),
      ],
      out_specs=pl.BlockSpec((_BLK, d_out), lambda i: (i, 0)),
      out_shape=jax.ShapeDtypeStruct((n_pad, d_out), jnp.float32),
  )(s2, y2, dinv, b2)


def kernel(x, edge_index, W1, b1, gamma, beta, W2, b2):
  n, d_in = x.shape
  d_hid = W1.shape[1]
  d_out = W2.shape[1]
  e = edge_index.shape[1]

  row_unit = _NS * _K
  n_pad = ((n + 64 + row_unit - 1) // row_unit) * row_unit
  e_unit = _NW * _K
  e_pad = ((e + e_unit - 1) // e_unit) * e_unit
  n_chunks = e_pad // (_NW * _K)

  pad = jnp.arange(e_pad - e, dtype=jnp.int32) % 64 + n
  src = jnp.concatenate([edge_index[0], pad]).reshape(_NW, n_chunks, _K)
  dst = jnp.concatenate([edge_index[1], pad]).reshape(_NW, n_chunks, _K)

  d2 = max(d_out, 128)
  w2p = jnp.pad(W2, ((0, 0), (0, d2 - d_out)))
  b2p = jnp.pad(b2, (0, d2 - d_out))

  xp = jnp.pad(x, ((0, n_pad - n), (0, 0)))
  b1r = b1.reshape(1, d_hid)
  b2r = b2.reshape(1, d_out)
  gr = gamma.reshape(1, d_hid)
  br = beta.reshape(1, d_hid)

  deg = _sc_degree(dst, n_pad, n_chunks).reshape(_NC, n_pad, 1)
  y1, dinv = _tc_y1(xp, W1, deg, n_pad, d_in, d_hid)
  s1 = _sc_scatter(y1, src, dst, n_pad, d_hid, n_chunks)
  h_pre, ssum, ssq = _tc_combine_stats(s1, y1, dinv, b1r, n, n_pad, d_hid)
  y2 = _tc_bn_relu_y2(h_pre, ssum, ssq, gr, br, w2p, dinv, n, n_pad, d_hid,
                      d2)
  s2 = _sc_scatter(y2, src, dst, n_pad, d2, n_chunks)
  out = _tc_finish(s2, y2, dinv, b2r, n_pad, d2, d_out)
  return out[:n]


# speedup vs baseline: 1.0100x; 1.0100x over previous
import functools

import jax
import jax.numpy as jnp
from jax import lax
from jax.experimental import pallas as pl
from jax.experimental.pallas import tpu as pltpu
from jax.experimental.pallas import tpu_sc as plsc

_NC = 2
_NS = 16
_L = 16
_NW = _NC * _NS
_K = 128
_BLK = 5120


def _sc_degree(dst_idx, n_pad, n_chunks):
  zrows = n_pad // _NS
  mesh = plsc.VectorSubcoreMesh(core_axis_name="c", subcore_axis_name="s")

  @functools.partial(
      pl.kernel,
      out_type=jax.ShapeDtypeStruct((_NC, n_pad), jnp.float32),
      mesh=mesh,
      scratch_types=[
          pltpu.VMEM((n_chunks, _K), jnp.int32),
          pltpu.VMEM((_K,), jnp.float32),
          pltpu.VMEM_SHARED((n_pad,), jnp.float32),
      ],
  )
  def k(dst_hbm, out_hbm, di_v, ones_v, deg_sh):
    cid = lax.axis_index("c")
    sid = lax.axis_index("s")
    wid = cid * _NS + sid
    pltpu.sync_copy(dst_hbm.at[wid], di_v)

    def _fill(val, i, carry):
      ones_v[pl.ds(i * _L, _L)] = jnp.full((_L,), val, jnp.float32)
      return carry

    lax.fori_loop(0, _K // _L, functools.partial(_fill, 0.0), 0)
    for z in range(zrows // _K):
      pltpu.sync_copy(ones_v, deg_sh.at[pl.ds(sid * zrows + z * _K, _K)])
    lax.fori_loop(0, _K // _L, functools.partial(_fill, 1.0), 0)
    plsc.subcore_barrier()

    def chunk(j, carry):
      pltpu.sync_copy(ones_v, deg_sh.at[di_v.at[j]], add=True)
      return carry

    lax.fori_loop(0, n_chunks, chunk, 0)
    plsc.subcore_barrier()
    pltpu.sync_copy(deg_sh.at[pl.ds(sid * zrows, zrows)],
                    out_hbm.at[cid, pl.ds(sid * zrows, zrows)])

  return k(dst_idx)


def _sc_scatter(y, src_idx, dst_idx, n_pad, d, n_chunks):
  zrows = n_pad // _NS
  mesh = plsc.VectorSubcoreMesh(core_axis_name="c", subcore_axis_name="s")

  @functools.partial(
      pl.kernel,
      out_type=jax.ShapeDtypeStruct((_NC, n_pad, d), jnp.float32),
      mesh=mesh,
      scratch_types=[
          pltpu.VMEM((n_chunks, _K), jnp.int32),
          pltpu.VMEM((n_chunks, _K), jnp.int32),
          pltpu.VMEM((_K, d), jnp.float32),
          pltpu.SemaphoreType.DMA,
          pltpu.VMEM_SHARED((n_pad, d), jnp.float32),
      ],
  )
  def k(y_hbm, src_hbm, dst_hbm, out_hbm, si_v, di_v, buf, gsem, s_sh):
    cid = lax.axis_index("c")
    sid = lax.axis_index("s")
    wid = cid * _NS + sid
    pltpu.sync_copy(src_hbm.at[wid], si_v)
    pltpu.sync_copy(dst_hbm.at[wid], di_v)

    def _zero(i, carry):
      for j in range(d // _L):
        buf[i, pl.ds(j * _L, _L)] = jnp.zeros((_L,), jnp.float32)
      return carry

    lax.fori_loop(0, _K, _zero, 0)
    for z in range(zrows // _K):
      pltpu.sync_copy(buf, s_sh.at[pl.ds(sid * zrows + z * _K, _K)])
    plsc.subcore_barrier()

    def chunk(j, carry):
      pltpu.async_copy(y_hbm.at[si_v.at[j]], buf, gsem).wait()
      pltpu.sync_copy(buf, s_sh.at[di_v.at[j]], add=True)
      return carry

    lax.fori_loop(0, n_chunks, chunk, 0)
    plsc.subcore_barrier()
    pltpu.sync_copy(s_sh.at[pl.ds(sid * zrows, zrows)],
                    out_hbm.at[cid, pl.ds(sid * zrows, zrows)])

  return k(y, src_idx, dst_idx)


def _tc_y1(x, w1, deg, n_pad, d_in, d_hid):
  def body(x_ref, w_ref, dg_ref, y_ref, di_ref):
    dg = dg_ref[...]
    dinv = lax.rsqrt(dg[0] + dg[1] + 1.0)
    di_ref[...] = dinv
    xw = jnp.dot(x_ref[...], w_ref[...], preferred_element_type=jnp.float32)
    y_ref[...] = xw * dinv

  return pl.pallas_call(
      body,
      grid=(n_pad // _BLK,),
      in_specs=[
          pl.BlockSpec((_BLK, d_in), lambda i: (i, 0)),
          pl.BlockSpec((d_in, d_hid), lambda i: (0, 0)),
          pl.BlockSpec((_NC, _BLK, 1), lambda i: (0, i, 0)),
      ],
      out_specs=[
          pl.BlockSpec((_BLK, d_hid), lambda i: (i, 0)),
          pl.BlockSpec((_BLK, 1), lambda i: (i, 0)),
      ],
      out_shape=[
          jax.ShapeDtypeStruct((n_pad, d_hid), jnp.float32),
          jax.ShapeDtypeStruct((n_pad, 1), jnp.float32),
      ],
  )(x, w1, deg)


def _tc_combine_stats(s, y1, dinv, b1, n_real, n_pad, d_hid):
  def body(s_ref, y_ref, di_ref, b_ref, h_ref, sum_ref, sq_ref):
    i = pl.program_id(0)
    h = di_ref[...] * (jnp.sum(s_ref[...], axis=0) + y_ref[...]) + b_ref[...]
    h_ref[...] = h
    rows = i * _BLK + lax.broadcasted_iota(jnp.int32, (_BLK, 1), 0)
    hm = jnp.where(rows < n_real, h, 0.0)

    @pl.when(i == 0)
    def _():
      sum_ref[...] = jnp.zeros_like(sum_ref)
      sq_ref[...] = jnp.zeros_like(sq_ref)

    sum_ref[...] += jnp.sum(hm, axis=0, keepdims=True)
    sq_ref[...] += jnp.sum(hm * hm, axis=0, keepdims=True)

  return pl.pallas_call(
      body,
      grid=(n_pad // _BLK,),
      in_specs=[
          pl.BlockSpec((_NC, _BLK, d_hid), lambda i: (0, i, 0)),
          pl.BlockSpec((_BLK, d_hid), lambda i: (i, 0)),
          pl.BlockSpec((_BLK, 1), lambda i: (i, 0)),
          pl.BlockSpec((1, d_hid), lambda i: (0, 0)),
      ],
      out_specs=[
          pl.BlockSpec((_BLK, d_hid), lambda i: (i, 0)),
          pl.BlockSpec((1, d_hid), lambda i: (0, 0)),
          pl.BlockSpec((1, d_hid), lambda i: (0, 0)),
      ],
      out_shape=[
          jax.ShapeDtypeStruct((n_pad, d_hid), jnp.float32),
          jax.ShapeDtypeStruct((1, d_hid), jnp.float32),
          jax.ShapeDtypeStruct((1, d_hid), jnp.float32),
      ],
  )(s, y1, dinv, b1)


def _tc_bn_relu_y2(h_pre, ssum, ssq, gamma, beta, w2, dinv, n_real, n_pad,
                   d_hid, d_out):
  def body(h_ref, sum_ref, sq_ref, g_ref, bt_ref, w_ref, di_ref, y_ref):
    mean = sum_ref[...] * (1.0 / n_real)
    var = sq_ref[...] * (1.0 / n_real) - mean * mean
    scale = lax.rsqrt(var + 1e-5) * g_ref[...]
    h = (h_ref[...] - mean) * scale + bt_ref[...]
    h = jnp.maximum(h, 0.0)
    hw = jnp.dot(h, w_ref[...], preferred_element_type=jnp.float32)
    y_ref[...] = hw * di_ref[...]

  return pl.pallas_call(
      body,
      grid=(n_pad // _BLK,),
      in_specs=[
          pl.BlockSpec((_BLK, d_hid), lambda i: (i, 0)),
          pl.BlockSpec((1, d_hid), lambda i: (0, 0)),
          pl.BlockSpec((1, d_hid), lambda i: (0, 0)),
          pl.BlockSpec((1, d_hid), lambda i: (0, 0)),
          pl.BlockSpec((1, d_hid), lambda i: (0, 0)),
          pl.BlockSpec((d_hid, d_out), lambda i: (0, 0)),
          pl.BlockSpec((_BLK, 1), lambda i: (i, 0)),
      ],
      out_specs=pl.BlockSpec((_BLK, d_out), lambda i: (i, 0)),
      out_shape=jax.ShapeDtypeStruct((n_pad, d_out), jnp.float32),
  )(h_pre, ssum, ssq, gamma, beta, w2, dinv)


def _tc_finish(s2, y2, dinv, b2, n_pad, d2, d_out):
  def body(s_ref, y_ref, di_ref, b_ref, o_ref):
    full = di_ref[...] * (jnp.sum(s_ref[...], axis=0) + y_ref[...])
    o_ref[...] = full[:, :d_out] + b_ref[...]

  return pl.pallas_call(
      body,
      grid=(n_pad // _BLK,),
      in_specs=[
          pl.BlockSpec((_NC, _BLK, d2), lambda i: (0, i, 0)),
          pl.BlockSpec((_BLK, d2), lambda i: (i, 0)),
          pl.BlockSpec((_BLK, 1), lambda i: (i, 0)),
          pl.BlockSpec((1, d_out), lambda i: (0, 0)),
      ],
      out_specs=pl.BlockSpec((_BLK, d_out), lambda i: (i, 0)),
      out_shape=jax.ShapeDtypeStruct((n_pad, d_out), jnp.float32),
  )(s2, y2, dinv, b2)


def kernel(x, edge_index, W1, b1, gamma, beta, W2, b2):
  n, d_in = x.shape
  d_hid = W1.shape[1]
  d_out = W2.shape[1]
  e = edge_index.shape[1]

  row_unit = _NS * _K
  n_pad = ((n + 64 + row_unit - 1) // row_unit) * row_unit
  e_unit = _NW * _K
  e_pad = ((e + e_unit - 1) // e_unit) * e_unit
  n_chunks = e_pad // (_NW * _K)

  pad = jnp.arange(e_pad - e, dtype=jnp.int32) % 64 + n
  src = jnp.concatenate([edge_index[0], pad]).reshape(_NW, n_chunks, _K)
  dst = jnp.concatenate([edge_index[1], pad]).reshape(_NW, n_chunks, _K)

  d2 = max(d_out, 128)
  w2p = jnp.pad(W2, ((0, 0), (0, d2 - d_out)))
  b2p = jnp.pad(b2, (0, d2 - d_out))

  xp = jnp.pad(x, ((0, n_pad - n), (0, 0)))
  b1r = b1.reshape(1, d_hid)
  b2r = b2.reshape(1, d_out)
  gr = gamma.reshape(1, d_hid)
  br = beta.reshape(1, d_hid)

  deg = _sc_degree(dst, n_pad, n_chunks).reshape(_NC, n_pad, 1)
  y1, dinv = _tc_y1(xp, W1, deg, n_pad, d_in, d_hid)
  s1 = _sc_scatter(y1, src, dst, n_pad, d_hid, n_chunks)
  h_pre, ssum, ssq = _tc_combine_stats(s1, y1, dinv, b1r, n, n_pad, d_hid)
  y2 = _tc_bn_relu_y2(h_pre, ssum, ssq, gr, br, w2p, dinv, n, n_pad, d_hid,
                      d2)
  s2 = _sc_scatter(y2, src, dst, n_pad, d2, n_chunks)
  out = _tc_finish(s2, y2, dinv, b2r, n_pad, d2, d_out)
  return out[:n]
